# baseline (speedup 1.0000x reference)
import functools

import jax
import jax.numpy as jnp
from jax import lax
from jax.experimental import pallas as pl
from jax.experimental.pallas import tpu as pltpu
from jax.experimental.pallas import tpu_sc as plsc

B = 16384
D = 64
DP = 128
NC = 2
NS = 16
NW = NC * NS
BPW = B // NW
CH = 128
NCH = BPW // CH
L = 16
PAD = L + 1

_mesh = plsc.VectorSubcoreMesh(core_axis_name="c", subcore_axis_name="s")


@functools.partial(
    pl.kernel,
    out_type=jax.ShapeDtypeStruct((B,), jnp.float32),
    mesh=_mesh,
    compiler_params=pltpu.CompilerParams(
        needs_layout_passes=False, use_tc_tiling_on_sc=True
    ),
    scratch_types=[
        pltpu.VMEM((2 * BPW,), jnp.int32),
        pltpu.VMEM((NCH, CH), jnp.int32),
        pltpu.VMEM((NCH, CH), jnp.int32),
        pltpu.VMEM((2, CH, DP), jnp.float32),
        pltpu.VMEM((2, CH, DP), jnp.float32),
        pltpu.VMEM((L * PAD,), jnp.float32),
        pltpu.VMEM((BPW,), jnp.float32),
        pltpu.SemaphoreType.DMA,
        pltpu.SemaphoreType.DMA,
        pltpu.SemaphoreType.DMA,
        pltpu.SemaphoreType.DMA,
    ],
)
def _sc_dual_gather_dot(pairs_hbm, user_hbm, item_hbm, out_hbm,
                        pairs_v, uidx_v, iidx_v, urows, irows, tmat, outv,
                        usem0, usem1, isem0, isem1):
    wid = lax.axis_index("s") * NC + lax.axis_index("c")
    base = wid * BPW

    pltpu.sync_copy(pairs_hbm.at[wid], pairs_v)
    iota = lax.iota(jnp.int32, L)
    iota2 = iota * 2
    for j in range(NCH):
        for k in range(CH // L):
            off = (j * CH + k * L) * 2
            uidx_v[j, pl.ds(k * L, L)] = plsc.load_gather(
                pairs_v, [iota2 + off])
            iidx_v[j, pl.ds(k * L, L)] = plsc.load_gather(
                pairs_v, [iota2 + (off + 1)])

    usems = [usem0, usem1]
    isems = [isem0, isem1]

    def fire(q):
        buf = q % 2
        cu = pltpu.async_copy(user_hbm.at[uidx_v.at[q]],
                              urows.at[buf], usems[buf])
        ci = pltpu.async_copy(item_hbm.at[iidx_v.at[q]],
                              irows.at[buf], isems[buf])
        return cu, ci

    gather_idx = [iota * PAD + l for l in range(L)]

    def compute_chunk(q):
        buf = q % 2

        def block_body(blk, _):
            rbase = blk * L
            for j in range(L):
                b = rbase + j
                s = (urows[buf, b, pl.ds(0, L)]
                     * irows[buf, b, pl.ds(0, L)])
                for d0 in range(L, D, L):
                    s = s + (urows[buf, b, pl.ds(d0, L)]
                             * irows[buf, b, pl.ds(d0, L)])
                tmat[pl.ds(j * PAD, L)] = s
            acc = plsc.load_gather(tmat, [gather_idx[0]])
            for l in range(1, L):
                acc = acc + plsc.load_gather(tmat, [gather_idx[l]])
            outv[pl.ds(q * CH + rbase, L)] = acc
            return 0

        lax.fori_loop(0, CH // L, block_body, 0)

    inflight = fire(0)
    for q in range(NCH):
        nxt = fire(q + 1) if q + 1 < NCH else None
        inflight[0].wait()
        inflight[1].wait()
        compute_chunk(q)
        inflight = nxt

    pltpu.sync_copy(outv, out_hbm.at[pl.ds(base, BPW)])


TR = 512


def _tr_body(src_ref, dst_ref):
    t = src_ref[...].T
    dst_ref[...] = jnp.concatenate(
        [t, jnp.zeros((TR, DP - D), jnp.float32)], axis=1)


_tc_detile = pl.pallas_call(
    _tr_body,
    grid=(pl.cdiv(100000, TR),),
    in_specs=[pl.BlockSpec((D, TR), lambda i: (0, i))],
    out_specs=pl.BlockSpec((TR, DP), lambda i: (i, 0)),
    out_shape=jax.ShapeDtypeStruct((100000, DP), jnp.float32),
)


def kernel(inputs, user_table, item_table):
    pairs = inputs.reshape(NW, 2 * BPW)
    up = _tc_detile(user_table.T)
    ip = _tc_detile(item_table.T)
    return _sc_dual_gather_dot(pairs, up, ip)


# speedup vs baseline: 1.8671x; 1.8671x over previous
import functools

import jax
import jax.numpy as jnp
from jax import lax
from jax.experimental import pallas as pl
from jax.experimental.pallas import tpu as pltpu
from jax.experimental.pallas import tpu_sc as plsc

B = 16384
D = 64
DP = 128
NC = 2
NS = 16
NW = NC * NS
BPW = B // NW
CH = 128
NCH = BPW // CH
L = 16
PAD = L + 1

_mesh = plsc.VectorSubcoreMesh(core_axis_name="c", subcore_axis_name="s")


@functools.partial(
    pl.kernel,
    out_type=jax.ShapeDtypeStruct((B,), jnp.float32),
    mesh=_mesh,
    compiler_params=pltpu.CompilerParams(
        needs_layout_passes=False, use_tc_tiling_on_sc=True
    ),
    scratch_types=[
        pltpu.VMEM((2 * BPW,), jnp.int32),
        pltpu.VMEM((NCH, CH), jnp.int32),
        pltpu.VMEM((NCH, CH), jnp.int32),
        pltpu.VMEM((NCH, CH), jnp.int32),
        pltpu.VMEM((NCH, CH), jnp.int32),
        pltpu.VMEM((2, CH, DP), jnp.float32),
        pltpu.VMEM((2, CH, DP), jnp.float32),
        pltpu.VMEM((L * PAD,), jnp.float32),
        pltpu.VMEM((L * PAD,), jnp.float32),
        pltpu.VMEM((L * PAD,), jnp.float32),
        pltpu.VMEM((L * PAD,), jnp.float32),
        pltpu.VMEM((BPW,), jnp.float32),
        pltpu.SemaphoreType.DMA,
        pltpu.SemaphoreType.DMA,
        pltpu.SemaphoreType.DMA,
        pltpu.SemaphoreType.DMA,
    ],
)
def _sc_dual_gather_dot(pairs_hbm, user_hbm, item_hbm, out_hbm,
                        pairs_v, uidx_v, iidx_v, uh_v, ih_v,
                        urows, irows, tAA, tAB, tBA, tBB, outv,
                        usem0, usem1, isem0, isem1):
    wid = lax.axis_index("s") * NC + lax.axis_index("c")
    base = wid * BPW

    pltpu.sync_copy(pairs_hbm.at[wid], pairs_v)
    iota = lax.iota(jnp.int32, L)
    iota2 = iota * 2
    one = jnp.full((L,), 1, jnp.int32)
    for j in range(NCH):
        for k in range(CH // L):
            off = (j * CH + k * L) * 2
            u = plsc.load_gather(pairs_v, [iota2 + off])
            i = plsc.load_gather(pairs_v, [iota2 + (off + 1)])
            sl = pl.ds(k * L, L)
            uidx_v[j, sl] = lax.shift_right_logical(u, one)
            iidx_v[j, sl] = lax.shift_right_logical(i, one)
            uh_v[j, sl] = lax.bitwise_and(u, one)
            ih_v[j, sl] = lax.bitwise_and(i, one)

    usems = [usem0, usem1]
    isems = [isem0, isem1]

    def fire(q):
        buf = q % 2
        cu = pltpu.async_copy(user_hbm.at[uidx_v.at[q]],
                              urows.at[buf], usems[buf])
        ci = pltpu.async_copy(item_hbm.at[iidx_v.at[q]],
                              irows.at[buf], isems[buf])
        return cu, ci

    gather_idx = [iota * PAD + l for l in range(L)]

    def compute_chunk(q):
        buf = q % 2

        def block_body(blk, _):
            rbase = blk * L
            for j in range(L):
                b = rbase + j
                sAA = sAB = sBA = sBB = None
                for d0 in range(0, D, L):
                    pa = urows[buf, b, pl.ds(d0, L)]
                    pb = urows[buf, b, pl.ds(D + d0, L)]
                    qa = irows[buf, b, pl.ds(d0, L)]
                    qb = irows[buf, b, pl.ds(D + d0, L)]
                    if sAA is None:
                        sAA, sAB, sBA, sBB = pa * qa, pa * qb, pb * qa, pb * qb
                    else:
                        sAA = sAA + pa * qa
                        sAB = sAB + pa * qb
                        sBA = sBA + pb * qa
                        sBB = sBB + pb * qb
                sl = pl.ds(j * PAD, L)
                tAA[sl] = sAA
                tAB[sl] = sAB
                tBA[sl] = sBA
                tBB[sl] = sBB
            accAA = plsc.load_gather(tAA, [gather_idx[0]])
            accAB = plsc.load_gather(tAB, [gather_idx[0]])
            accBA = plsc.load_gather(tBA, [gather_idx[0]])
            accBB = plsc.load_gather(tBB, [gather_idx[0]])
            for l in range(1, L):
                accAA = accAA + plsc.load_gather(tAA, [gather_idx[l]])
                accAB = accAB + plsc.load_gather(tAB, [gather_idx[l]])
                accBA = accBA + plsc.load_gather(tBA, [gather_idx[l]])
                accBB = accBB + plsc.load_gather(tBB, [gather_idx[l]])
            hu = uh_v[q, pl.ds(rbase, L)] > 0
            hi = ih_v[q, pl.ds(rbase, L)] > 0
            acc = jnp.where(hu, jnp.where(hi, accBB, accBA),
                            jnp.where(hi, accAB, accAA))
            outv[pl.ds(q * CH + rbase, L)] = acc
            return 0

        lax.fori_loop(0, CH // L, block_body, 0)

    inflight = fire(0)
    for q in range(NCH):
        nxt = fire(q + 1) if q + 1 < NCH else None
        inflight[0].wait()
        inflight[1].wait()
        compute_chunk(q)
        inflight = nxt

    pltpu.sync_copy(outv, out_hbm.at[pl.ds(base, BPW)])


def kernel(inputs, user_table, item_table):
    pairs = inputs.reshape(NW, 2 * BPW)
    u2 = user_table.reshape(50000, DP)
    i2 = item_table.reshape(50000, DP)
    return _sc_dual_gather_dot(pairs, u2, i2)


# speedup vs baseline: 2.2434x; 1.2015x over previous
import functools

import jax
import jax.numpy as jnp
from jax import lax
from jax.experimental import pallas as pl
from jax.experimental.pallas import tpu as pltpu
from jax.experimental.pallas import tpu_sc as plsc

B = 16384
D = 64
NC = 2
NS = 16
NW = NC * NS
BPW = B // NW
CH = 128
NCH = BPW // CH
L = 16
PAD = L + 1

_mesh = plsc.VectorSubcoreMesh(core_axis_name="c", subcore_axis_name="s")


@functools.partial(
    pl.kernel,
    out_type=jax.ShapeDtypeStruct((B,), jnp.float32),
    mesh=_mesh,
    compiler_params=pltpu.CompilerParams(
        needs_layout_passes=False, use_tc_tiling_on_sc=True
    ),
    scratch_types=[
        pltpu.VMEM((2 * BPW,), jnp.int32),
        pltpu.VMEM((NCH, CH), jnp.int32),
        pltpu.VMEM((NCH, CH), jnp.int32),
        pltpu.VMEM((2, CH, D), jnp.float32),
        pltpu.VMEM((2, CH, D), jnp.float32),
        pltpu.VMEM((L * PAD,), jnp.float32),
        pltpu.VMEM((BPW,), jnp.float32),
        pltpu.SemaphoreType.DMA,
        pltpu.SemaphoreType.DMA,
        pltpu.SemaphoreType.DMA,
        pltpu.SemaphoreType.DMA,
    ],
)
def _sc_dual_gather_dot(pairs_hbm, user_hbm, item_hbm, out_hbm,
                        pairs_v, uidx_v, iidx_v, urows, irows, tmat, outv,
                        usem0, usem1, isem0, isem1):
    wid = lax.axis_index("s") * NC + lax.axis_index("c")
    base = wid * BPW

    pltpu.sync_copy(pairs_hbm.at[wid], pairs_v)
    iota = lax.iota(jnp.int32, L)
    iota2 = iota * 2
    for j in range(NCH):
        for k in range(CH // L):
            off = (j * CH + k * L) * 2
            uidx_v[j, pl.ds(k * L, L)] = plsc.load_gather(
                pairs_v, [iota2 + off])
            iidx_v[j, pl.ds(k * L, L)] = plsc.load_gather(
                pairs_v, [iota2 + (off + 1)])

    usems = [usem0, usem1]
    isems = [isem0, isem1]

    def fire_group(q, buf, g):
        k0 = g * L
        uvec = uidx_v[q, pl.ds(k0, L)]
        ivec = iidx_v[q, pl.ds(k0, L)]
        for j in range(L):
            pltpu.async_copy(user_hbm.at[pl.ds(uvec[j], 1), :],
                             urows.at[buf, pl.ds(k0 + j, 1), :], usems[buf])
            pltpu.async_copy(item_hbm.at[pl.ds(ivec[j], 1), :],
                             irows.at[buf, pl.ds(k0 + j, 1), :], isems[buf])

    def drain_group(buf):
        for j in range(L):
            pltpu.make_async_copy(user_hbm.at[pl.ds(0, 1), :],
                                  urows.at[buf, pl.ds(j, 1), :],
                                  usems[buf]).wait()
            pltpu.make_async_copy(item_hbm.at[pl.ds(0, 1), :],
                                  irows.at[buf, pl.ds(j, 1), :],
                                  isems[buf]).wait()

    NG = CH // L

    def fire(q):
        buf = q % 2
        fire_group(q, buf, 0)

        def body(g, _):
            fire_group(q, buf, g)
            drain_group(buf)
            return 0

        lax.fori_loop(1, NG, body, 0)
        return buf

    def drain_tail(buf):
        drain_group(buf)

    gather_idx = [iota * PAD + l for l in range(L)]

    def compute_chunk(q):
        buf = q % 2

        def block_body(blk, _):
            rbase = blk * L
            for j in range(L):
                b = rbase + j
                s = (urows[buf, b, pl.ds(0, L)]
                     * irows[buf, b, pl.ds(0, L)])
                for d0 in range(L, D, L):
                    s = s + (urows[buf, b, pl.ds(d0, L)]
                             * irows[buf, b, pl.ds(d0, L)])
                tmat[pl.ds(j * PAD, L)] = s
            acc = plsc.load_gather(tmat, [gather_idx[0]])
            for l in range(1, L):
                acc = acc + plsc.load_gather(tmat, [gather_idx[l]])
            outv[pl.ds(q * CH + rbase, L)] = acc
            return 0

        lax.fori_loop(0, CH // L, block_body, 0)

    inflight = fire(0)
    for q in range(NCH):
        nxt = fire(q + 1) if q + 1 < NCH else None
        drain_tail(inflight)
        compute_chunk(q)
        inflight = nxt

    pltpu.sync_copy(outv, out_hbm.at[pl.ds(base, BPW)])


def kernel(inputs, user_table, item_table):
    pairs = inputs.reshape(NW, 2 * BPW)
    return _sc_dual_gather_dot(pairs, user_table, item_table)


# speedup vs baseline: 2.5585x; 1.1404x over previous
import functools

import jax
import jax.numpy as jnp
from jax import lax
from jax.experimental import pallas as pl
from jax.experimental.pallas import tpu as pltpu
from jax.experimental.pallas import tpu_sc as plsc

B = 16384
D = 64
NC = 2
NS = 16
NW = NC * NS
BPW = B // NW
CH = 128
NCH = BPW // CH
L = 16
PAD = L + 1

_mesh = plsc.VectorSubcoreMesh(core_axis_name="c", subcore_axis_name="s")


@functools.partial(
    pl.kernel,
    out_type=jax.ShapeDtypeStruct((B,), jnp.float32),
    mesh=_mesh,
    compiler_params=pltpu.CompilerParams(
        needs_layout_passes=False, use_tc_tiling_on_sc=True
    ),
    scratch_types=[
        pltpu.VMEM((BPW,), jnp.int32),
        pltpu.VMEM((BPW,), jnp.int32),
        pltpu.VMEM((2, CH, D), jnp.float32),
        pltpu.VMEM((2, CH, D), jnp.float32),
        pltpu.VMEM((L * PAD,), jnp.float32),
        pltpu.VMEM((BPW,), jnp.float32),
        pltpu.SemaphoreType.DMA,
        pltpu.SemaphoreType.DMA,
        pltpu.SemaphoreType.DMA,
        pltpu.SemaphoreType.DMA,
    ],
)
def _sc_dual_gather_dot(idx_hbm, user_hbm, item_hbm, out_hbm,
                        uix, iix, urows, irows, tmat, outv,
                        usem0, usem1, isem0, isem1):
    wid = lax.axis_index("s") * NC + lax.axis_index("c")
    base = wid * BPW

    pltpu.sync_copy(idx_hbm.at[0, pl.ds(base, BPW)], uix)
    pltpu.sync_copy(idx_hbm.at[1, pl.ds(base, BPW)], iix)

    usems = [usem0, usem1]
    isems = [isem0, isem1]

    def fire_group(q, buf, g):
        k0 = g * L
        uvec = uix[pl.ds(q * CH + k0, L)]
        ivec = iix[pl.ds(q * CH + k0, L)]
        for j in range(L):
            pltpu.async_copy(user_hbm.at[pl.ds(uvec[j], 1), :],
                             urows.at[buf, pl.ds(k0 + j, 1), :], usems[buf])
            pltpu.async_copy(item_hbm.at[pl.ds(ivec[j], 1), :],
                             irows.at[buf, pl.ds(k0 + j, 1), :], isems[buf])

    def drain_group(buf):
        for j in range(L):
            pltpu.make_async_copy(user_hbm.at[pl.ds(0, 1), :],
                                  urows.at[buf, pl.ds(j, 1), :],
                                  usems[buf]).wait()
            pltpu.make_async_copy(item_hbm.at[pl.ds(0, 1), :],
                                  irows.at[buf, pl.ds(j, 1), :],
                                  isems[buf]).wait()

    NG = CH // L

    def fire(q):
        buf = q % 2
        fire_group(q, buf, 0)
        fire_group(q, buf, 1)

        def body(g, _):
            fire_group(q, buf, g)
            drain_group(buf)
            return 0

        lax.fori_loop(2, NG, body, 0)
        return buf

    def drain_tail(buf):
        drain_group(buf)
        drain_group(buf)

    iota = lax.iota(jnp.int32, L)
    gather_idx = [iota * PAD + l for l in range(L)]

    def compute_chunk(q):
        buf = q % 2

        def block_body(blk, _):
            rbase = blk * L
            for j in range(L):
                b = rbase + j
                s = (urows[buf, b, pl.ds(0, L)]
                     * irows[buf, b, pl.ds(0, L)])
                for d0 in range(L, D, L):
                    s = s + (urows[buf, b, pl.ds(d0, L)]
                             * irows[buf, b, pl.ds(d0, L)])
                tmat[pl.ds(j * PAD, L)] = s
            acc = plsc.load_gather(tmat, [gather_idx[0]])
            for l in range(1, L):
                acc = acc + plsc.load_gather(tmat, [gather_idx[l]])
            outv[pl.ds(q * CH + rbase, L)] = acc
            return 0

        lax.fori_loop(0, CH // L, block_body, 0)

    buf = fire(0)
    for q in range(NCH):
        nxt = fire(q + 1) if q + 1 < NCH else None
        drain_tail(buf)
        compute_chunk(q)
        buf = nxt

    pltpu.sync_copy(outv, out_hbm.at[pl.ds(base, BPW)])


def kernel(inputs, user_table, item_table):
    return _sc_dual_gather_dot(inputs.T, user_table, item_table)


# speedup vs baseline: 2.6818x; 1.0482x over previous
import functools

import jax
import jax.numpy as jnp
from jax import lax
from jax.experimental import pallas as pl
from jax.experimental.pallas import tpu as pltpu
from jax.experimental.pallas import tpu_sc as plsc

B = 16384
D = 64
NC = 2
NS = 16
NW = NC * NS
BPW = B // NW
CH = 128
NCH = BPW // CH
L = 16
PAD = L + 1

_mesh = plsc.VectorSubcoreMesh(core_axis_name="c", subcore_axis_name="s")


@functools.partial(
    pl.kernel,
    out_type=jax.ShapeDtypeStruct((B,), jnp.float32),
    mesh=_mesh,
    compiler_params=pltpu.CompilerParams(
        needs_layout_passes=False, use_tc_tiling_on_sc=True
    ),
    scratch_types=[
        pltpu.VMEM((BPW,), jnp.int32),
        pltpu.VMEM((BPW,), jnp.int32),
        pltpu.VMEM((2, CH, D), jnp.float32),
        pltpu.VMEM((2, CH, D), jnp.float32),
        pltpu.VMEM((L * PAD,), jnp.float32),
        pltpu.VMEM((BPW,), jnp.float32),
        pltpu.SemaphoreType.DMA,
        pltpu.SemaphoreType.DMA,
        pltpu.SemaphoreType.DMA,
        pltpu.SemaphoreType.DMA,
    ],
)
def _sc_dual_gather_dot(idx_hbm, user_hbm, item_hbm, out_hbm,
                        uix, iix, urows, irows, tmat, outv,
                        usem0, usem1, isem0, isem1):
    wid = lax.axis_index("s") * NC + lax.axis_index("c")
    base = wid * BPW

    pltpu.sync_copy(idx_hbm.at[0, pl.ds(base, BPW)], uix)
    pltpu.sync_copy(idx_hbm.at[1, pl.ds(base, BPW)], iix)

    usems = [usem0, usem1]
    isems = [isem0, isem1]

    def fire_group(q, buf, g):
        k0 = g * L
        uvec = uix[pl.ds(q * CH + k0, L)]
        ivec = iix[pl.ds(q * CH + k0, L)]
        for j in range(L):
            pltpu.async_copy(user_hbm.at[pl.ds(uvec[j], 1), :],
                             urows.at[buf, pl.ds(k0 + j, 1), :], usems[buf])
            pltpu.async_copy(item_hbm.at[pl.ds(ivec[j], 1), :],
                             irows.at[buf, pl.ds(k0 + j, 1), :], isems[buf])

    def drain_group(buf):
        pltpu.make_async_copy(user_hbm.at[pl.ds(0, L), :],
                              urows.at[buf, pl.ds(0, L), :],
                              usems[buf]).wait()
        pltpu.make_async_copy(item_hbm.at[pl.ds(0, L), :],
                              irows.at[buf, pl.ds(0, L), :],
                              isems[buf]).wait()

    NG = CH // L
    DEPTH = 4

    def fire(q):
        buf = q % 2
        for g in range(DEPTH):
            fire_group(q, buf, g)

        def body(g, _):
            fire_group(q, buf, g)
            drain_group(buf)
            return 0

        lax.fori_loop(DEPTH, NG, body, 0)
        return buf

    def drain_tail(buf):
        for _ in range(DEPTH):
            drain_group(buf)

    iota = lax.iota(jnp.int32, L)
    gather_idx = [iota * PAD + l for l in range(L)]

    def compute_chunk(q):
        buf = q % 2

        def block_body(blk, _):
            rbase = blk * L
            for j in range(L):
                b = rbase + j
                s = (urows[buf, b, pl.ds(0, L)]
                     * irows[buf, b, pl.ds(0, L)])
                for d0 in range(L, D, L):
                    s = s + (urows[buf, b, pl.ds(d0, L)]
                             * irows[buf, b, pl.ds(d0, L)])
                tmat[pl.ds(j * PAD, L)] = s
            acc = plsc.load_gather(tmat, [gather_idx[0]])
            for l in range(1, L):
                acc = acc + plsc.load_gather(tmat, [gather_idx[l]])
            outv[pl.ds(q * CH + rbase, L)] = acc
            return 0

        lax.fori_loop(0, CH // L, block_body, 0)

    buf = fire(0)
    for q in range(NCH):
        nxt = fire(q + 1) if q + 1 < NCH else None
        drain_tail(buf)
        compute_chunk(q)
        buf = nxt

    pltpu.sync_copy(outv, out_hbm.at[pl.ds(base, BPW)])


def kernel(inputs, user_table, item_table):
    return _sc_dual_gather_dot(inputs.T, user_table, item_table)


# speedup vs baseline: 2.8057x; 1.0462x over previous
import functools

import jax
import jax.numpy as jnp
from jax import lax
from jax.experimental import pallas as pl
from jax.experimental.pallas import tpu as pltpu
from jax.experimental.pallas import tpu_sc as plsc

B = 16384
D = 64
NC = 2
NS = 16
NW = NC * NS
BPW = B // NW
CH = 128
NCH = BPW // CH
L = 16
PAD = L + 1

_mesh = plsc.VectorSubcoreMesh(core_axis_name="c", subcore_axis_name="s")


@functools.partial(
    pl.kernel,
    out_type=jax.ShapeDtypeStruct((B,), jnp.float32),
    mesh=_mesh,
    compiler_params=pltpu.CompilerParams(
        needs_layout_passes=False, use_tc_tiling_on_sc=True
    ),
    scratch_types=[
        pltpu.VMEM((BPW,), jnp.int32),
        pltpu.VMEM((BPW,), jnp.int32),
        pltpu.VMEM((2, CH, D), jnp.float32),
        pltpu.VMEM((2, CH, D), jnp.float32),
        pltpu.VMEM((L * PAD,), jnp.float32),
        pltpu.VMEM((BPW,), jnp.float32),
        pltpu.SemaphoreType.DMA,
        pltpu.SemaphoreType.DMA,
        pltpu.SemaphoreType.DMA,
        pltpu.SemaphoreType.DMA,
    ],
)
def _sc_dual_gather_dot(idx_hbm, user_hbm, item_hbm, out_hbm,
                        uix, iix, urows, irows, tmat, outv,
                        usem0, usem1, isem0, isem1):
    wid = lax.axis_index("s") * NC + lax.axis_index("c")
    base = wid * BPW

    pltpu.sync_copy(idx_hbm.at[0, pl.ds(base, BPW)], uix)
    pltpu.sync_copy(idx_hbm.at[1, pl.ds(base, BPW)], iix)

    usems = [usem0, usem1]
    isems = [isem0, isem1]

    def fire_group(q, buf, g):
        k0 = g * L
        uvec = uix[pl.ds(q * CH + k0, L)]
        ivec = iix[pl.ds(q * CH + k0, L)]
        for j in range(L):
            pltpu.async_copy(user_hbm.at[pl.ds(uvec[j], 1), :],
                             urows.at[buf, pl.ds(k0 + j, 1), :], usems[buf])
            pltpu.async_copy(item_hbm.at[pl.ds(ivec[j], 1), :],
                             irows.at[buf, pl.ds(k0 + j, 1), :], isems[buf])

    def drain_group(buf):
        pltpu.make_async_copy(user_hbm.at[pl.ds(0, L), :],
                              urows.at[buf, pl.ds(0, L), :],
                              usems[buf]).wait()
        pltpu.make_async_copy(item_hbm.at[pl.ds(0, L), :],
                              irows.at[buf, pl.ds(0, L), :],
                              isems[buf]).wait()

    NG = CH // L

    def fire(q):
        buf = q % 2

        def body(g, _):
            fire_group(q, buf, g)
            return 0

        lax.fori_loop(0, NG, body, 0)
        return buf

    def drain_tail(buf):
        for _ in range(NG):
            drain_group(buf)

    iota = lax.iota(jnp.int32, L)
    gather_idx = [iota * PAD + l for l in range(L)]

    def compute_chunk(q):
        buf = q % 2

        def block_body(blk, _):
            rbase = blk * L
            for j in range(L):
                b = rbase + j
                s = (urows[buf, b, pl.ds(0, L)]
                     * irows[buf, b, pl.ds(0, L)])
                for d0 in range(L, D, L):
                    s = s + (urows[buf, b, pl.ds(d0, L)]
                             * irows[buf, b, pl.ds(d0, L)])
                tmat[pl.ds(j * PAD, L)] = s
            acc = plsc.load_gather(tmat, [gather_idx[0]])
            for l in range(1, L):
                acc = acc + plsc.load_gather(tmat, [gather_idx[l]])
            outv[pl.ds(q * CH + rbase, L)] = acc
            return 0

        lax.fori_loop(0, CH // L, block_body, 0)

    buf = fire(0)
    for q in range(NCH):
        nxt = fire(q + 1) if q + 1 < NCH else None
        drain_tail(buf)
        compute_chunk(q)
        buf = nxt

    pltpu.sync_copy(outv, out_hbm.at[pl.ds(base, BPW)])


def kernel(inputs, user_table, item_table):
    return _sc_dual_gather_dot(inputs.T, user_table, item_table)
